# baseline (speedup 1.0000x reference)
import functools

import jax
import jax.numpy as jnp
import numpy as np
from jax import lax
from jax.experimental import pallas as pl
from jax.experimental.pallas import tpu as pltpu
from jax.experimental.pallas import tpu_sc as plsc

D = 32
B = 16384

NC = 2
NS = 16
NW = NC * NS
BPW = B // NW
CHUNK = 128
NCHUNK = BPW // CHUNK


def _body(uidx_hbm, iidx_hbm, upw, ucw, ipw, icw,
          rating_out, up_out, uc_out, ip_out, ic_out,
          uidx_v, iidx_v, up_v, uc_v, ip_v, ic_v, logit_v,
          gsem, osem):
    wid = lax.axis_index("s") * NC + lax.axis_index("c")
    base = pl.multiple_of(wid * BPW, BPW)

    pltpu.sync_copy(uidx_hbm.at[pl.ds(base, BPW)], uidx_v)
    pltpu.sync_copy(iidx_hbm.at[pl.ds(base, BPW)], iidx_v)

    copies = []
    for c in range(NCHUNK):
        sl = pl.ds(c * CHUNK, CHUNK)
        copies.append(pltpu.async_copy(upw.at[uidx_v.at[sl]], up_v.at[sl], gsem))
        copies.append(pltpu.async_copy(ucw.at[uidx_v.at[sl]], uc_v.at[sl], gsem))
        copies.append(pltpu.async_copy(ipw.at[iidx_v.at[sl]], ip_v.at[sl], gsem))
        copies.append(pltpu.async_copy(icw.at[iidx_v.at[sl]], ic_v.at[sl], gsem))
    for cp in copies:
        cp.wait()

    row_sl = pl.ds(base, BPW)
    outs = [
        pltpu.async_copy(up_v, up_out.at[row_sl], osem),
        pltpu.async_copy(uc_v, uc_out.at[row_sl], osem),
        pltpu.async_copy(ip_v, ip_out.at[row_sl], osem),
        pltpu.async_copy(ic_v, ic_out.at[row_sl], osem),
    ]

    lo = pl.ds(0, 16)
    hi = pl.ds(16, 16)
    lane = lax.iota(jnp.int32, 16)
    lane_masks = [lane == j for j in range(16)]

    def group_body(g, carry):
        rbase = pl.multiple_of(g * 16, 16)
        logits = jnp.zeros((16,), jnp.float32)
        for j in range(16):
            r = rbase + j
            u0 = up_v[r, lo] + uc_v[r, lo]
            u1 = up_v[r, hi] + uc_v[r, hi]
            i0 = ip_v[r, lo] + ic_v[r, lo]
            i1 = ip_v[r, hi] + ic_v[r, hi]
            s = jnp.sum(u0 * i0 + u1 * i1)
            logits = jnp.where(lane_masks[j], s, logits)
        logit_v[pl.ds(rbase, 16)] = 1.0 / (1.0 + jnp.exp(-logits))
        return carry

    lax.fori_loop(0, BPW // 16, group_body, 0)

    pltpu.sync_copy(logit_v, rating_out.at[row_sl])
    for o in outs:
        o.wait()


_fed_dad = functools.partial(
    pl.kernel,
    out_type=(
        jax.ShapeDtypeStruct((B,), jnp.float32),
        jax.ShapeDtypeStruct((B, D), jnp.float32),
        jax.ShapeDtypeStruct((B, D), jnp.float32),
        jax.ShapeDtypeStruct((B, D), jnp.float32),
        jax.ShapeDtypeStruct((B, D), jnp.float32),
    ),
    scratch_types=[
        pltpu.VMEM((BPW,), jnp.int32),
        pltpu.VMEM((BPW,), jnp.int32),
        pltpu.VMEM((BPW, D), jnp.float32),
        pltpu.VMEM((BPW, D), jnp.float32),
        pltpu.VMEM((BPW, D), jnp.float32),
        pltpu.VMEM((BPW, D), jnp.float32),
        pltpu.VMEM((BPW,), jnp.float32),
        pltpu.SemaphoreType.DMA,
        pltpu.SemaphoreType.DMA,
    ],
    mesh=plsc.VectorSubcoreMesh(core_axis_name="c", subcore_axis_name="s"),
    compiler_params=pltpu.CompilerParams(
        needs_layout_passes=False, use_tc_tiling_on_sc=False),
)(_body)


def kernel(user_indices, item_indices, user_personality_w, user_commonality_w,
           item_personality_w, item_commonality_w):
    rating, up, uc, ip, ic = _fed_dad(
        user_indices.astype(jnp.int32), item_indices.astype(jnp.int32),
        user_personality_w, user_commonality_w,
        item_personality_w, item_commonality_w)
    return (rating.reshape(B, 1), up, uc, ip, ic)


# speedup vs baseline: 2.9487x; 2.9487x over previous
import functools

import jax
import jax.numpy as jnp
from jax import lax
from jax.experimental import pallas as pl
from jax.experimental.pallas import tpu as pltpu
from jax.experimental.pallas import tpu_sc as plsc

D = 32
B = 16384

NC = 2
NS = 16
NW = NC * NS
BPW = B // NW


def _body(uidx_hbm, iidx_hbm, upw, ucw, ipw, icw,
          rating_out, up_out, uc_out, ip_out, ic_out,
          uidx_v, iidx_v,
          wup0, wuc0, wip0, wic0, wup1, wuc1, wip1, wic1,
          sup, suc, sip, sic, slg,
          sem0, sem1, osem):
    wid = lax.axis_index("s") * NC + lax.axis_index("c")
    base = pl.multiple_of(wid * BPW, BPW)

    pltpu.sync_copy(uidx_hbm.at[pl.ds(base, BPW)], uidx_v.at[pl.ds(0, BPW)])
    pltpu.sync_copy(iidx_hbm.at[pl.ds(base, BPW)], iidx_v.at[pl.ds(0, BPW)])

    wins = ((wup0, wuc0, wip0, wic0), (wup1, wuc1, wip1, wic1))
    sems = (sem0, sem1)
    d0 = lax.iota(jnp.int32, 16)
    d1 = d0 + 16
    lane0 = d0 == 0

    def fetch(j, p):
        jc = jnp.minimum(j, BPW - 1)
        ru = uidx_v[pl.ds(jc, 16)][0]
        ri = iidx_v[pl.ds(jc, 16)][0]
        wu = pl.multiple_of(lax.shift_left(lax.shift_right_logical(ru, 7), 7), 128)
        wi = pl.multiple_of(lax.shift_left(lax.shift_right_logical(ri, 7), 7), 128)
        wu_sl = pl.ds(wu, 128)
        wi_sl = pl.ds(wi, 128)
        w = wins[p]
        pltpu.make_async_copy(upw.at[:, wu_sl], w[0], sems[p]).start()
        pltpu.make_async_copy(ucw.at[:, wu_sl], w[1], sems[p]).start()
        pltpu.make_async_copy(ipw.at[:, wi_sl], w[2], sems[p]).start()
        pltpu.make_async_copy(icw.at[:, wi_sl], w[3], sems[p]).start()

    def drain(p):
        w = wins[p]
        head = pl.ds(0, 128)
        pltpu.make_async_copy(upw.at[:, head], w[0], sems[p]).wait()
        pltpu.make_async_copy(ucw.at[:, head], w[1], sems[p]).wait()
        pltpu.make_async_copy(ipw.at[:, head], w[2], sems[p]).wait()
        pltpu.make_async_copy(icw.at[:, head], w[3], sems[p]).wait()

    def extract(j, p):
        ru = uidx_v[pl.ds(j, 16)][0]
        ri = iidx_v[pl.ds(j, 16)][0]
        cu = jnp.broadcast_to(ru & 127, (16,))
        ci = jnp.broadcast_to(ri & 127, (16,))
        w = wins[p]
        up0 = plsc.load_gather(w[0], [d0, cu])
        up1 = plsc.load_gather(w[0], [d1, cu])
        uc0 = plsc.load_gather(w[1], [d0, cu])
        uc1 = plsc.load_gather(w[1], [d1, cu])
        ip0 = plsc.load_gather(w[2], [d0, ci])
        ip1 = plsc.load_gather(w[2], [d1, ci])
        ic0 = plsc.load_gather(w[3], [d0, ci])
        ic1 = plsc.load_gather(w[3], [d1, ci])
        off = pl.multiple_of(j * D, D)
        off2 = pl.multiple_of(j * D + 16, 16)
        sup[pl.ds(off, 16)] = up0
        sup[pl.ds(off2, 16)] = up1
        suc[pl.ds(off, 16)] = uc0
        suc[pl.ds(off2, 16)] = uc1
        sip[pl.ds(off, 16)] = ip0
        sip[pl.ds(off2, 16)] = ip1
        sic[pl.ds(off, 16)] = ic0
        sic[pl.ds(off2, 16)] = ic1
        s = jnp.sum((up0 + uc0) * (ip0 + ic0) + (up1 + uc1) * (ip1 + ic1))
        plsc.store_scatter(slg, [jnp.broadcast_to(j, (16,))],
                           jnp.broadcast_to(s, (16,)), mask=lane0)

    fetch(0, 0)

    def body(k, carry):
        j0 = k * 2
        fetch(j0 + 1, 1)
        drain(0)
        extract(j0, 0)
        fetch(j0 + 2, 0)
        drain(1)
        extract(j0 + 1, 1)
        return carry

    lax.fori_loop(0, BPW // 2, body, 0)
    drain(0)

    def sig_body(g, carry):
        sl = pl.ds(pl.multiple_of(g * 16, 16), 16)
        x = slg[sl]
        slg[sl] = 1.0 / (1.0 + jnp.exp(-x))
        return carry

    lax.fori_loop(0, BPW // 16, sig_body, 0)

    obase = pl.multiple_of(base * D, BPW * D)
    outs = [pltpu.async_copy(sup, up_out.at[pl.ds(obase, BPW * D)], osem),
            pltpu.async_copy(suc, uc_out.at[pl.ds(obase, BPW * D)], osem),
            pltpu.async_copy(sip, ip_out.at[pl.ds(obase, BPW * D)], osem),
            pltpu.async_copy(sic, ic_out.at[pl.ds(obase, BPW * D)], osem)]
    pltpu.sync_copy(slg, rating_out.at[pl.ds(base, BPW)])
    for o in outs:
        o.wait()


_fed_dad = functools.partial(
    pl.kernel,
    out_type=(
        jax.ShapeDtypeStruct((B,), jnp.float32),
        jax.ShapeDtypeStruct((B * D,), jnp.float32),
        jax.ShapeDtypeStruct((B * D,), jnp.float32),
        jax.ShapeDtypeStruct((B * D,), jnp.float32),
        jax.ShapeDtypeStruct((B * D,), jnp.float32),
    ),
    scratch_types=[
        pltpu.VMEM((BPW + 16,), jnp.int32),
        pltpu.VMEM((BPW + 16,), jnp.int32),
    ] + [pltpu.VMEM((D, 128), jnp.float32)] * 8 + [
        pltpu.VMEM((BPW * D,), jnp.float32),
        pltpu.VMEM((BPW * D,), jnp.float32),
        pltpu.VMEM((BPW * D,), jnp.float32),
        pltpu.VMEM((BPW * D,), jnp.float32),
        pltpu.VMEM((BPW,), jnp.float32),
        pltpu.SemaphoreType.DMA,
        pltpu.SemaphoreType.DMA,
        pltpu.SemaphoreType.DMA,
    ],
    mesh=plsc.VectorSubcoreMesh(core_axis_name="c", subcore_axis_name="s"),
    compiler_params=pltpu.CompilerParams(
        needs_layout_passes=False, use_tc_tiling_on_sc=True),
)(_body)


def kernel(user_indices, item_indices, user_personality_w, user_commonality_w,
           item_personality_w, item_commonality_w):
    rating, up, uc, ip, ic = _fed_dad(
        user_indices.astype(jnp.int32), item_indices.astype(jnp.int32),
        user_personality_w.T, user_commonality_w.T,
        item_personality_w.T, item_commonality_w.T)
    return (rating.reshape(B, 1), up.reshape(B, D), uc.reshape(B, D),
            ip.reshape(B, D), ic.reshape(B, D))


# speedup vs baseline: 3.5974x; 1.2200x over previous
import functools

import jax
import jax.numpy as jnp
from jax import lax
from jax.experimental import pallas as pl
from jax.experimental.pallas import tpu as pltpu
from jax.experimental.pallas import tpu_sc as plsc

D = 32
B = 16384

NC = 2
NS = 16
NW = NC * NS
BPW = B // NW


RING = 4
CHUNK = 128


def _body(uidx_hbm, iidx_hbm, upw, ucw, ipw, icw,
          rating_out, up_out, uc_out, ip_out, ic_out,
          uidx_v, iidx_v,
          wup0, wuc0, wip0, wic0, wup1, wuc1, wip1, wic1,
          wup2, wuc2, wip2, wic2, wup3, wuc3, wip3, wic3,
          sup, suc, sip, sic, slg,
          sem0, sem1, sem2, sem3):
    wid = lax.axis_index("s") * NC + lax.axis_index("c")
    base = pl.multiple_of(wid * BPW, BPW)

    pltpu.sync_copy(uidx_hbm.at[pl.ds(base, BPW)], uidx_v.at[pl.ds(0, BPW)])
    pltpu.sync_copy(iidx_hbm.at[pl.ds(base, BPW)], iidx_v.at[pl.ds(0, BPW)])

    wins = ((wup0, wuc0, wip0, wic0), (wup1, wuc1, wip1, wic1),
            (wup2, wuc2, wip2, wic2), (wup3, wuc3, wip3, wic3))
    sems = (sem0, sem1, sem2, sem3)
    d0 = lax.iota(jnp.int32, 16)
    d1 = d0 + 16
    lane0 = d0 == 0

    def fetch(j, p):
        jc = jnp.minimum(j, BPW - 1)
        ru = uidx_v[pl.ds(jc, 16)][0]
        ri = iidx_v[pl.ds(jc, 16)][0]
        wu = pl.multiple_of(lax.shift_left(lax.shift_right_logical(ru, 7), 7), 128)
        wi = pl.multiple_of(lax.shift_left(lax.shift_right_logical(ri, 7), 7), 128)
        wu_sl = pl.ds(wu, 128)
        wi_sl = pl.ds(wi, 128)
        w = wins[p]
        pltpu.make_async_copy(upw.at[:, wu_sl], w[0], sems[p]).start()
        pltpu.make_async_copy(ucw.at[:, wu_sl], w[1], sems[p]).start()
        pltpu.make_async_copy(ipw.at[:, wi_sl], w[2], sems[p]).start()
        pltpu.make_async_copy(icw.at[:, wi_sl], w[3], sems[p]).start()

    def drain(p):
        w = wins[p]
        head = pl.ds(0, 128)
        pltpu.make_async_copy(upw.at[:, head], w[0], sems[p]).wait()
        pltpu.make_async_copy(ucw.at[:, head], w[1], sems[p]).wait()
        pltpu.make_async_copy(ipw.at[:, head], w[2], sems[p]).wait()
        pltpu.make_async_copy(icw.at[:, head], w[3], sems[p]).wait()

    def extract(j, p):
        ru = uidx_v[pl.ds(j, 16)][0]
        ri = iidx_v[pl.ds(j, 16)][0]
        cu = jnp.broadcast_to(ru & 127, (16,))
        ci = jnp.broadcast_to(ri & 127, (16,))
        w = wins[p]
        up0 = plsc.load_gather(w[0], [d0, cu])
        up1 = plsc.load_gather(w[0], [d1, cu])
        uc0 = plsc.load_gather(w[1], [d0, cu])
        uc1 = plsc.load_gather(w[1], [d1, cu])
        ip0 = plsc.load_gather(w[2], [d0, ci])
        ip1 = plsc.load_gather(w[2], [d1, ci])
        ic0 = plsc.load_gather(w[3], [d0, ci])
        ic1 = plsc.load_gather(w[3], [d1, ci])
        off = pl.multiple_of((j & (CHUNK - 1)) * D, D)
        off2 = pl.multiple_of((j & (CHUNK - 1)) * D + 16, 16)
        sup[pl.ds(off, 16)] = up0
        sup[pl.ds(off2, 16)] = up1
        suc[pl.ds(off, 16)] = uc0
        suc[pl.ds(off2, 16)] = uc1
        sip[pl.ds(off, 16)] = ip0
        sip[pl.ds(off2, 16)] = ip1
        sic[pl.ds(off, 16)] = ic0
        sic[pl.ds(off2, 16)] = ic1
        s = jnp.sum((up0 + uc0) * (ip0 + ic0) + (up1 + uc1) * (ip1 + ic1))
        plsc.store_scatter(slg, [jnp.broadcast_to(j, (16,))],
                           jnp.broadcast_to(s, (16,)), mask=lane0)

    def flush(c):
        dst = pl.multiple_of(base * D + c * CHUNK * D, CHUNK * D)
        dsl = pl.ds(dst, CHUNK * D)
        pltpu.sync_copy(sup, up_out.at[dsl])
        pltpu.sync_copy(suc, uc_out.at[dsl])
        pltpu.sync_copy(sip, ip_out.at[dsl])
        pltpu.sync_copy(sic, ic_out.at[dsl])

    for p in range(RING):
        fetch(p, p)

    def body(k, carry):
        for q in range(RING):
            j = k * RING + q
            drain(q)
            extract(j, q)
            fetch(j + RING, q)

        @pl.when((k & 31) == 31)
        def _():
            flush(k >> 5)

        return carry

    lax.fori_loop(0, BPW // RING, body, 0)
    for p in range(RING):
        drain(p)

    def sig_body(g, carry):
        sl = pl.ds(pl.multiple_of(g * 16, 16), 16)
        x = slg[sl]
        slg[sl] = 1.0 / (1.0 + jnp.exp(-x))
        return carry

    lax.fori_loop(0, BPW // 16, sig_body, 0)

    pltpu.sync_copy(slg, rating_out.at[pl.ds(base, BPW)])


_fed_dad = functools.partial(
    pl.kernel,
    out_type=(
        jax.ShapeDtypeStruct((B,), jnp.float32),
        jax.ShapeDtypeStruct((B * D,), jnp.float32),
        jax.ShapeDtypeStruct((B * D,), jnp.float32),
        jax.ShapeDtypeStruct((B * D,), jnp.float32),
        jax.ShapeDtypeStruct((B * D,), jnp.float32),
    ),
    scratch_types=[
        pltpu.VMEM((BPW + 16,), jnp.int32),
        pltpu.VMEM((BPW + 16,), jnp.int32),
    ] + [pltpu.VMEM((D, 128), jnp.float32)] * (4 * RING) + [
        pltpu.VMEM((CHUNK * D,), jnp.float32),
        pltpu.VMEM((CHUNK * D,), jnp.float32),
        pltpu.VMEM((CHUNK * D,), jnp.float32),
        pltpu.VMEM((CHUNK * D,), jnp.float32),
        pltpu.VMEM((BPW,), jnp.float32),
    ] + [pltpu.SemaphoreType.DMA] * RING,
    mesh=plsc.VectorSubcoreMesh(core_axis_name="c", subcore_axis_name="s"),
    compiler_params=pltpu.CompilerParams(
        needs_layout_passes=False, use_tc_tiling_on_sc=True),
)(_body)


def kernel(user_indices, item_indices, user_personality_w, user_commonality_w,
           item_personality_w, item_commonality_w):
    rating, up, uc, ip, ic = _fed_dad(
        user_indices.astype(jnp.int32), item_indices.astype(jnp.int32),
        user_personality_w.T, user_commonality_w.T,
        item_personality_w.T, item_commonality_w.T)
    return (rating.reshape(B, 1), up.reshape(B, D), uc.reshape(B, D),
            ip.reshape(B, D), ic.reshape(B, D))
